# baseline (speedup 1.0000x reference)
import functools

import jax
import jax.numpy as jnp
from jax import lax
from jax.experimental import pallas as pl
from jax.experimental.pallas import tpu as pltpu
from jax.experimental.pallas import tpu_sc as plsc

_L = 16
_BIG = 1e6
_TAU = 28.0 / 1024.0
_STRIDE = 48
_RPB = 8
_INF_I = 2**30


def _greedy_body(dist_hbm, params_hbm, penalty_hbm, predinit_hbm, out_hbm,
                 sval, sidx, counts_v, vis_v, pred_v, row_v, prm_v, bufs,
                 sems):
    n = dist_hbm.shape[1]
    nchunks = n // _L
    nblk = n // _RPB
    c = lax.axis_index("c")
    s = lax.axis_index("s")
    b = s * 2 + c

    lanes = lax.iota(jnp.int32, _L)
    big_v = jnp.full((_L,), _BIG, jnp.float32)
    zero_iv = jnp.zeros((_L,), jnp.int32)

    pltpu.sync_copy(params_hbm.at[b], prm_v)
    prm = prm_v[...]
    start = jnp.max(jnp.where(lanes == 0, prm, 0))
    limit = jnp.max(jnp.where(lanes == 1, prm, 0))

    pltpu.sync_copy(penalty_hbm.at[b], vis_v)
    pltpu.sync_copy(predinit_hbm.at[b], pred_v)

    tau = jnp.float32(_TAU)
    pltpu.async_copy(dist_hbm.at[b, pl.ds(0, _RPB)], bufs.at[0], sems.at[0])

    @plsc.parallel_loop(0, n * _STRIDE // _L, unroll=8)
    def _prefill(k):
        sval[pl.ds(k * _L, _L)] = big_v
        sidx[pl.ds(k * _L, _L)] = zero_iv

    def blk2(g2, _):
        for half in range(2):
            g = g2 * 2 + half

            @pl.when(g + 1 < nblk)
            def _issue():
                pltpu.async_copy(
                    dist_hbm.at[b, pl.ds((g + 1) * _RPB, _RPB)],
                    bufs.at[1 - half], sems.at[1 - half])

            pltpu.make_async_copy(dist_hbm.at[b, pl.ds(g * _RPB, _RPB)],
                                  bufs.at[half], sems.at[half]).wait()

            def row(rr, _):
                rowid = g * _RPB + rr
                cb = rowid * _STRIDE
                base = jnp.full((_L,), cb - 1, jnp.uint32)
                cap = jnp.full((_L,), cb + _STRIDE - 1, jnp.uint32)

                ones_uv = jnp.ones((_L,), jnp.uint32)

                @plsc.parallel_loop(0, nchunks, unroll=4,
                                    carry=jnp.zeros((_L,), jnp.uint32))
                def woff_v(k, wv):
                    off = k * _L
                    v = bufs[half, rr, pl.ds(off, _L)]
                    m = v < tau
                    pfx = plsc.cumsum(ones_uv, mask=m)
                    pos = plsc.bitcast(jnp.minimum(base + wv + pfx, cap),
                                       jnp.int32)
                    plsc.store_scatter(sval, [pos], v, mask=m)
                    plsc.store_scatter(sidx, [pos], lanes + off, mask=m)
                    return wv + plsc.bitcast(
                        plsc.all_reduce_population_count(m), jnp.uint32)

                plsc.store_scatter(
                    counts_v, [jnp.full((_L,), rowid, jnp.int32)],
                    plsc.bitcast(woff_v, jnp.int32), mask=lanes == 0)
                return 0

            lax.fori_loop(0, _RPB, row, 0)
        return 0

    lax.fori_loop(0, nblk // 2, blk2, 0)

    def of_sweep(k, _):
        cnt = counts_v[pl.ds(k * _L, _L)]
        nof = plsc.all_reduce_population_count(cnt > _STRIDE)[0]

        @pl.when(nof > 0)
        def _fix():
            for l in range(_L):
                @pl.when(cnt[l] > _STRIDE)
                def _fill():
                    cb2 = (k * _L + l) * _STRIDE
                    for c3 in range(_STRIDE // _L):
                        sval[pl.ds(cb2 + c3 * _L, _L)] = big_v

        return 0

    lax.fori_loop(0, n // _L, of_sweep, 0)

    def step(j, point):
        cb = point * _STRIDE

        bv = jnp.full((_L,), _BIG, jnp.float32)
        bi = jnp.full((_L,), _INF_I, jnp.int32)
        for c2 in range(_STRIDE // _L):
            sv = sval[pl.ds(cb + c2 * _L, _L)]
            si = sidx[pl.ds(cb + c2 * _L, _L)]
            g = plsc.load_gather(vis_v, [si])
            v = jnp.where(g == 0.0, sv, jnp.float32(_BIG))
            lt = v < bv
            bv = jnp.where(lt, v, bv)
            bi = jnp.where(lt, si, bi)
        m = jnp.min(bv)
        idx_f = jnp.min(jnp.where(bv == m, bi, _INF_I))

        def fast():
            return idx_f

        def slow():
            pltpu.sync_copy(dist_hbm.at[b, point], row_v)

            @plsc.parallel_loop(0, nchunks, unroll=8,
                                carry=(jnp.full((_L,), 3e6, jnp.float32),
                                       jnp.zeros((_L,), jnp.int32)))
            def fcarry(k, carry):
                fv, fi = carry
                off = k * _L
                v = row_v[pl.ds(off, _L)]
                p = vis_v[pl.ds(off, _L)]
                v = jnp.where(p != 0.0, jnp.float32(_BIG), v)
                flt = v < fv
                return (jnp.where(flt, v, fv),
                        jnp.where(flt, lanes + off, fi))

            fv, fi = fcarry
            fm = jnp.min(fv)
            return jnp.min(jnp.where(fv == fm, fi, _INF_I))

        idx = lax.cond(idx_f < _INF_I, fast, slow)

        idx_vec = jnp.full((_L,), idx, jnp.int32)
        lane0 = lanes == 0
        plsc.store_scatter(vis_v, [idx_vec], big_v, mask=lane0)
        wr = jnp.logical_and(lane0, j < limit)
        plsc.store_scatter(pred_v, [jnp.full((_L,), j, jnp.int32)], idx_vec,
                           mask=wr)
        return idx

    lax.fori_loop(0, n, step, start)
    pltpu.sync_copy(pred_v, out_hbm.at[b])


def kernel(distance, mask, start_idx, pad_value):
    B, N, _ = distance.shape
    assert B == 32 and N % _L == 0 and N % _RPB == 0

    penalty = jnp.where(mask, jnp.float32(_BIG), jnp.float32(0.0))
    limit = (N - jnp.sum(mask.astype(jnp.int32), axis=1)).astype(jnp.int32)
    params = jnp.zeros((B, _L), jnp.int32)
    params = params.at[:, 0].set(start_idx.astype(jnp.int32))
    params = params.at[:, 1].set(limit)
    predinit = jnp.full((B, N), pad_value, jnp.int32)

    mesh = plsc.VectorSubcoreMesh(core_axis_name="c", subcore_axis_name="s")
    run = pl.kernel(
        _greedy_body,
        out_type=jax.ShapeDtypeStruct((B, N), jnp.int32),
        mesh=mesh,
        compiler_params=pltpu.CompilerParams(needs_layout_passes=False),
        scratch_types=[
            pltpu.VMEM((N * _STRIDE,), jnp.float32),
            pltpu.VMEM((N * _STRIDE,), jnp.int32),
            pltpu.VMEM((N,), jnp.int32),
            pltpu.VMEM((N,), jnp.float32),
            pltpu.VMEM((N,), jnp.int32),
            pltpu.VMEM((N,), jnp.float32),
            pltpu.VMEM((_L,), jnp.int32),
            pltpu.VMEM((2, _RPB, N), jnp.float32),
            pltpu.SemaphoreType.DMA((2,)),
        ],
    )
    preds = run(distance, params, penalty, predinit)
    return preds, limit


# speedup vs baseline: 1.0840x; 1.0840x over previous
import functools

import jax
import jax.numpy as jnp
from jax import lax
from jax.experimental import pallas as pl
from jax.experimental.pallas import tpu as pltpu
from jax.experimental.pallas import tpu_sc as plsc

_L = 16
_BIG = 1e6
_TAU = 28.0 / 1024.0
_STRIDE = 48
_RPB = 8
_INF_I = 2**30


def _greedy_body(dist_hbm, params_hbm, penalty_hbm, predinit_hbm, out_hbm,
                 sval, sidx, counts_v, vis_v, pred_v, row_v, prm_v, bufs,
                 sems):
    n = dist_hbm.shape[1]
    nchunks = n // _L
    nblk = n // _RPB
    c = lax.axis_index("c")
    s = lax.axis_index("s")
    b = s * 2 + c

    lanes = lax.iota(jnp.int32, _L)
    big_v = jnp.full((_L,), _BIG, jnp.float32)
    zero_iv = jnp.zeros((_L,), jnp.int32)

    pltpu.sync_copy(params_hbm.at[b], prm_v)
    prm = prm_v[...]
    start = jnp.max(jnp.where(lanes == 0, prm, 0))
    limit = jnp.max(jnp.where(lanes == 1, prm, 0))

    pltpu.sync_copy(penalty_hbm.at[b], vis_v)
    pltpu.sync_copy(predinit_hbm.at[b], pred_v)

    tau = jnp.float32(_TAU)
    pltpu.async_copy(dist_hbm.at[b, pl.ds(0, _RPB)], bufs.at[0], sems.at[0])

    @plsc.parallel_loop(0, n * _STRIDE // _L, unroll=8)
    def _prefill(k):
        sval[pl.ds(k * _L, _L)] = big_v
        sidx[pl.ds(k * _L, _L)] = zero_iv

    def blk2(g2, _):
        for half in range(2):
            g = g2 * 2 + half

            @pl.when(g + 1 < nblk)
            def _issue():
                pltpu.async_copy(
                    dist_hbm.at[b, pl.ds((g + 1) * _RPB, _RPB)],
                    bufs.at[1 - half], sems.at[1 - half])

            pltpu.make_async_copy(dist_hbm.at[b, pl.ds(g * _RPB, _RPB)],
                                  bufs.at[half], sems.at[half]).wait()

            def row(rr, _):
                rowid = g * _RPB + rr
                cb = rowid * _STRIDE
                base = jnp.full((_L,), cb - 1, jnp.uint32)
                cap = jnp.full((_L,), cb + _STRIDE - 1, jnp.uint32)

                ones_uv = jnp.ones((_L,), jnp.uint32)

                @plsc.parallel_loop(0, nchunks, unroll=8,
                                    carry=jnp.zeros((_L,), jnp.uint32))
                def woff_v(k, wv):
                    off = k * _L
                    v = bufs[half, rr, pl.ds(off, _L)]
                    m = v < tau
                    pfx = plsc.cumsum(ones_uv, mask=m)
                    pos = plsc.bitcast(jnp.minimum(base + wv + pfx, cap),
                                       jnp.int32)
                    plsc.store_scatter(sval, [pos], v, mask=m)
                    plsc.store_scatter(sidx, [pos], lanes + off, mask=m)
                    return wv + plsc.bitcast(
                        plsc.all_reduce_population_count(m), jnp.uint32)

                plsc.store_scatter(
                    counts_v, [jnp.full((_L,), rowid, jnp.int32)],
                    plsc.bitcast(woff_v, jnp.int32), mask=lanes == 0)
                return 0

            lax.fori_loop(0, _RPB, row, 0)
        return 0

    lax.fori_loop(0, nblk // 2, blk2, 0)

    def of_sweep(k, _):
        cnt = counts_v[pl.ds(k * _L, _L)]
        nof = plsc.all_reduce_population_count(cnt > _STRIDE)[0]

        @pl.when(nof > 0)
        def _fix():
            for l in range(_L):
                @pl.when(cnt[l] > _STRIDE)
                def _fill():
                    cb2 = (k * _L + l) * _STRIDE
                    for c3 in range(_STRIDE // _L):
                        sval[pl.ds(cb2 + c3 * _L, _L)] = big_v

        return 0

    lax.fori_loop(0, n // _L, of_sweep, 0)

    def step(j, point):
        cb = point * _STRIDE

        bv = jnp.full((_L,), _BIG, jnp.float32)
        bi = jnp.full((_L,), _INF_I, jnp.int32)
        for c2 in range(_STRIDE // _L):
            sv = sval[pl.ds(cb + c2 * _L, _L)]
            si = sidx[pl.ds(cb + c2 * _L, _L)]
            g = plsc.load_gather(vis_v, [si])
            v = jnp.where(g == 0.0, sv, jnp.float32(_BIG))
            lt = v < bv
            bv = jnp.where(lt, v, bv)
            bi = jnp.where(lt, si, bi)
        m = jnp.min(bv)
        idx_f = jnp.min(jnp.where(bv == m, bi, _INF_I))

        def fast():
            return idx_f

        def slow():
            pltpu.sync_copy(dist_hbm.at[b, point], row_v)

            @plsc.parallel_loop(0, nchunks, unroll=8,
                                carry=(jnp.full((_L,), 3e6, jnp.float32),
                                       jnp.zeros((_L,), jnp.int32)))
            def fcarry(k, carry):
                fv, fi = carry
                off = k * _L
                v = row_v[pl.ds(off, _L)]
                p = vis_v[pl.ds(off, _L)]
                v = jnp.where(p != 0.0, jnp.float32(_BIG), v)
                flt = v < fv
                return (jnp.where(flt, v, fv),
                        jnp.where(flt, lanes + off, fi))

            fv, fi = fcarry
            fm = jnp.min(fv)
            return jnp.min(jnp.where(fv == fm, fi, _INF_I))

        idx = lax.cond(idx_f < _INF_I, fast, slow)

        idx_vec = jnp.full((_L,), idx, jnp.int32)
        lane0 = lanes == 0
        plsc.store_scatter(vis_v, [idx_vec], big_v, mask=lane0)
        wr = jnp.logical_and(lane0, j < limit)
        plsc.store_scatter(pred_v, [jnp.full((_L,), j, jnp.int32)], idx_vec,
                           mask=wr)
        return idx

    lax.fori_loop(0, n, step, start)
    pltpu.sync_copy(pred_v, out_hbm.at[b])


def kernel(distance, mask, start_idx, pad_value):
    B, N, _ = distance.shape
    assert B == 32 and N % _L == 0 and N % _RPB == 0

    penalty = jnp.where(mask, jnp.float32(_BIG), jnp.float32(0.0))
    limit = (N - jnp.sum(mask.astype(jnp.int32), axis=1)).astype(jnp.int32)
    params = jnp.zeros((B, _L), jnp.int32)
    params = params.at[:, 0].set(start_idx.astype(jnp.int32))
    params = params.at[:, 1].set(limit)
    predinit = jnp.full((B, N), pad_value, jnp.int32)

    mesh = plsc.VectorSubcoreMesh(core_axis_name="c", subcore_axis_name="s")
    run = pl.kernel(
        _greedy_body,
        out_type=jax.ShapeDtypeStruct((B, N), jnp.int32),
        mesh=mesh,
        compiler_params=pltpu.CompilerParams(needs_layout_passes=False),
        scratch_types=[
            pltpu.VMEM((N * _STRIDE,), jnp.float32),
            pltpu.VMEM((N * _STRIDE,), jnp.int32),
            pltpu.VMEM((N,), jnp.int32),
            pltpu.VMEM((N,), jnp.float32),
            pltpu.VMEM((N,), jnp.int32),
            pltpu.VMEM((N,), jnp.float32),
            pltpu.VMEM((_L,), jnp.int32),
            pltpu.VMEM((2, _RPB, N), jnp.float32),
            pltpu.SemaphoreType.DMA((2,)),
        ],
    )
    preds = run(distance, params, penalty, predinit)
    return preds, limit


# speedup vs baseline: 1.0982x; 1.0131x over previous
import functools

import jax
import jax.numpy as jnp
from jax import lax
from jax.experimental import pallas as pl
from jax.experimental.pallas import tpu as pltpu
from jax.experimental.pallas import tpu_sc as plsc

_L = 16
_BIG = 1e6
_TAU = 32.0 / 1024.0
_STRIDE = 48
_RPB = 8
_INF_I = 2**30


def _greedy_body(dist_hbm, params_hbm, penalty_hbm, predinit_hbm, out_hbm,
                 sval, sidx, counts_v, vis_v, pred_v, row_v, prm_v, bufs,
                 sems):
    n = dist_hbm.shape[1]
    nchunks = n // _L
    nblk = n // _RPB
    c = lax.axis_index("c")
    s = lax.axis_index("s")
    b = s * 2 + c

    lanes = lax.iota(jnp.int32, _L)
    big_v = jnp.full((_L,), _BIG, jnp.float32)
    zero_iv = jnp.zeros((_L,), jnp.int32)

    pltpu.sync_copy(params_hbm.at[b], prm_v)
    prm = prm_v[...]
    start = jnp.max(jnp.where(lanes == 0, prm, 0))
    limit = jnp.max(jnp.where(lanes == 1, prm, 0))

    pltpu.sync_copy(penalty_hbm.at[b], vis_v)
    pltpu.sync_copy(predinit_hbm.at[b], pred_v)

    tau = jnp.float32(_TAU)
    pltpu.async_copy(dist_hbm.at[b, pl.ds(0, _RPB)], bufs.at[0], sems.at[0])

    @plsc.parallel_loop(0, n * _STRIDE // _L, unroll=8)
    def _prefill(k):
        sval[pl.ds(k * _L, _L)] = big_v
        sidx[pl.ds(k * _L, _L)] = zero_iv

    def blk2(g2, _):
        for half in range(2):
            g = g2 * 2 + half

            @pl.when(g + 1 < nblk)
            def _issue():
                pltpu.async_copy(
                    dist_hbm.at[b, pl.ds((g + 1) * _RPB, _RPB)],
                    bufs.at[1 - half], sems.at[1 - half])

            pltpu.make_async_copy(dist_hbm.at[b, pl.ds(g * _RPB, _RPB)],
                                  bufs.at[half], sems.at[half]).wait()

            def row(rr, _):
                rowid = g * _RPB + rr
                cb = rowid * _STRIDE
                base = jnp.full((_L,), cb - 1, jnp.uint32)
                cap = jnp.full((_L,), cb + _STRIDE - 1, jnp.uint32)

                ones_uv = jnp.ones((_L,), jnp.uint32)

                @plsc.parallel_loop(0, nchunks, unroll=8,
                                    carry=jnp.zeros((_L,), jnp.uint32))
                def woff_v(k, wv):
                    off = k * _L
                    v = bufs[half, rr, pl.ds(off, _L)]
                    m = v < tau
                    pfx = plsc.cumsum(ones_uv, mask=m)
                    pos = plsc.bitcast(jnp.minimum(base + wv + pfx, cap),
                                       jnp.int32)
                    plsc.store_scatter(sval, [pos], v, mask=m)
                    plsc.store_scatter(sidx, [pos], lanes + off, mask=m)
                    return wv + plsc.bitcast(
                        plsc.all_reduce_population_count(m), jnp.uint32)

                plsc.store_scatter(
                    counts_v, [jnp.full((_L,), rowid, jnp.int32)],
                    plsc.bitcast(woff_v, jnp.int32), mask=lanes == 0)
                return 0

            lax.fori_loop(0, _RPB, row, 0)
        return 0

    lax.fori_loop(0, nblk // 2, blk2, 0)

    def of_sweep(k, _):
        cnt = counts_v[pl.ds(k * _L, _L)]
        nof = plsc.all_reduce_population_count(cnt > _STRIDE)[0]

        @pl.when(nof > 0)
        def _fix():
            for l in range(_L):
                @pl.when(cnt[l] > _STRIDE)
                def _fill():
                    cb2 = (k * _L + l) * _STRIDE
                    for c3 in range(_STRIDE // _L):
                        sval[pl.ds(cb2 + c3 * _L, _L)] = big_v

        return 0

    lax.fori_loop(0, n // _L, of_sweep, 0)

    def step(j, point):
        cb = point * _STRIDE

        bv = jnp.full((_L,), _BIG, jnp.float32)
        bi = jnp.full((_L,), _INF_I, jnp.int32)
        for c2 in range(_STRIDE // _L):
            sv = sval[pl.ds(cb + c2 * _L, _L)]
            si = sidx[pl.ds(cb + c2 * _L, _L)]
            g = plsc.load_gather(vis_v, [si])
            v = jnp.where(g == 0.0, sv, jnp.float32(_BIG))
            lt = v < bv
            bv = jnp.where(lt, v, bv)
            bi = jnp.where(lt, si, bi)
        m = jnp.min(bv)
        idx_f = jnp.min(jnp.where(bv == m, bi, _INF_I))

        def fast():
            return idx_f

        def slow():
            pltpu.sync_copy(dist_hbm.at[b, point], row_v)

            @plsc.parallel_loop(0, nchunks, unroll=8,
                                carry=(jnp.full((_L,), 3e6, jnp.float32),
                                       jnp.zeros((_L,), jnp.int32)))
            def fcarry(k, carry):
                fv, fi = carry
                off = k * _L
                v = row_v[pl.ds(off, _L)]
                p = vis_v[pl.ds(off, _L)]
                v = jnp.where(p != 0.0, jnp.float32(_BIG), v)
                flt = v < fv
                return (jnp.where(flt, v, fv),
                        jnp.where(flt, lanes + off, fi))

            fv, fi = fcarry
            fm = jnp.min(fv)
            return jnp.min(jnp.where(fv == fm, fi, _INF_I))

        idx = lax.cond(idx_f < _INF_I, fast, slow)

        idx_vec = jnp.full((_L,), idx, jnp.int32)
        lane0 = lanes == 0
        plsc.store_scatter(vis_v, [idx_vec], big_v, mask=lane0)
        wr = jnp.logical_and(lane0, j < limit)
        plsc.store_scatter(pred_v, [jnp.full((_L,), j, jnp.int32)], idx_vec,
                           mask=wr)
        return idx

    lax.fori_loop(0, n, step, start)
    pltpu.sync_copy(pred_v, out_hbm.at[b])


def kernel(distance, mask, start_idx, pad_value):
    B, N, _ = distance.shape
    assert B == 32 and N % _L == 0 and N % _RPB == 0

    penalty = jnp.where(mask, jnp.float32(_BIG), jnp.float32(0.0))
    limit = (N - jnp.sum(mask.astype(jnp.int32), axis=1)).astype(jnp.int32)
    params = jnp.zeros((B, _L), jnp.int32)
    params = params.at[:, 0].set(start_idx.astype(jnp.int32))
    params = params.at[:, 1].set(limit)
    predinit = jnp.full((B, N), pad_value, jnp.int32)

    mesh = plsc.VectorSubcoreMesh(core_axis_name="c", subcore_axis_name="s")
    run = pl.kernel(
        _greedy_body,
        out_type=jax.ShapeDtypeStruct((B, N), jnp.int32),
        mesh=mesh,
        compiler_params=pltpu.CompilerParams(needs_layout_passes=False),
        scratch_types=[
            pltpu.VMEM((N * _STRIDE,), jnp.float32),
            pltpu.VMEM((N * _STRIDE,), jnp.int32),
            pltpu.VMEM((N,), jnp.int32),
            pltpu.VMEM((N,), jnp.float32),
            pltpu.VMEM((N,), jnp.int32),
            pltpu.VMEM((N,), jnp.float32),
            pltpu.VMEM((_L,), jnp.int32),
            pltpu.VMEM((2, _RPB, N), jnp.float32),
            pltpu.SemaphoreType.DMA((2,)),
        ],
    )
    preds = run(distance, params, penalty, predinit)
    return preds, limit


# speedup vs baseline: 1.0999x; 1.0016x over previous
import functools

import jax
import jax.numpy as jnp
from jax import lax
from jax.experimental import pallas as pl
from jax.experimental.pallas import tpu as pltpu
from jax.experimental.pallas import tpu_sc as plsc

_L = 16
_BIG = 1e6
_TAU = 32.0 / 1024.0
_STRIDE = 48
_RPB = 8
_INF_I = 2**30


def _greedy_body(dist_hbm, params_hbm, penalty_hbm, predinit_hbm, out_hbm,
                 sval, sidx, counts_v, vis_v, pred_v, row_v, prm_v, bufs,
                 sems):
    n = dist_hbm.shape[1]
    nchunks = n // _L
    nblk = n // _RPB
    c = lax.axis_index("c")
    s = lax.axis_index("s")
    b = s * 2 + c

    lanes = lax.iota(jnp.int32, _L)
    big_v = jnp.full((_L,), _BIG, jnp.float32)
    zero_iv = jnp.zeros((_L,), jnp.int32)

    pltpu.sync_copy(params_hbm.at[b], prm_v)
    prm = prm_v[...]
    start = jnp.max(jnp.where(lanes == 0, prm, 0))
    limit = jnp.max(jnp.where(lanes == 1, prm, 0))

    pltpu.sync_copy(penalty_hbm.at[b], vis_v)
    pltpu.sync_copy(predinit_hbm.at[b], pred_v)

    tau = jnp.float32(_TAU)
    pltpu.async_copy(dist_hbm.at[b, pl.ds(0, _RPB)], bufs.at[0], sems.at[0])

    @plsc.parallel_loop(0, n * _STRIDE // _L, unroll=8)
    def _prefill(k):
        sval[pl.ds(k * _L, _L)] = big_v
        sidx[pl.ds(k * _L, _L)] = zero_iv

    def blk2(g2, _):
        for half in range(2):
            g = g2 * 2 + half

            @pl.when(g + 1 < nblk)
            def _issue():
                pltpu.async_copy(
                    dist_hbm.at[b, pl.ds((g + 1) * _RPB, _RPB)],
                    bufs.at[1 - half], sems.at[1 - half])

            pltpu.make_async_copy(dist_hbm.at[b, pl.ds(g * _RPB, _RPB)],
                                  bufs.at[half], sems.at[half]).wait()

            def row(rr, _):
                rowid = g * _RPB + rr
                cb = rowid * _STRIDE
                base = jnp.full((_L,), cb - 1, jnp.uint32)
                cap = jnp.full((_L,), cb + _STRIDE - 1, jnp.uint32)

                ones_uv = jnp.ones((_L,), jnp.uint32)

                @plsc.parallel_loop(0, nchunks, unroll=8,
                                    carry=jnp.zeros((_L,), jnp.uint32))
                def woff_v(k, wv):
                    off = k * _L
                    v = bufs[half, rr, pl.ds(off, _L)]
                    m = v < tau
                    pfx = plsc.cumsum(ones_uv, mask=m)
                    pos = plsc.bitcast(jnp.minimum(base + wv + pfx, cap),
                                       jnp.int32)
                    plsc.store_scatter(sval, [pos], v, mask=m)
                    plsc.store_scatter(sidx, [pos], lanes + off, mask=m)
                    return wv + plsc.bitcast(
                        plsc.all_reduce_population_count(m), jnp.uint32)

                plsc.store_scatter(
                    counts_v, [jnp.full((_L,), rowid, jnp.int32)],
                    plsc.bitcast(woff_v, jnp.int32), mask=lanes == 0)
                return 0

            lax.fori_loop(0, _RPB, row, 0)
        return 0

    lax.fori_loop(0, nblk // 2, blk2, 0)

    def of_sweep(k, _):
        cnt = counts_v[pl.ds(k * _L, _L)]
        nof = plsc.all_reduce_population_count(cnt > _STRIDE)[0]

        @pl.when(nof > 0)
        def _fix():
            for l in range(_L):
                @pl.when(cnt[l] > _STRIDE)
                def _fill():
                    cb2 = (k * _L + l) * _STRIDE
                    for c3 in range(_STRIDE // _L):
                        sval[pl.ds(cb2 + c3 * _L, _L)] = big_v

        return 0

    lax.fori_loop(0, n // _L, of_sweep, 0)

    def step(j, point):
        cb = point * _STRIDE

        bv = jnp.full((_L,), _BIG, jnp.float32)
        bi = jnp.full((_L,), _INF_I, jnp.int32)
        for c2 in range(_STRIDE // _L):
            sv = sval[pl.ds(cb + c2 * _L, _L)]
            si = sidx[pl.ds(cb + c2 * _L, _L)]
            g = plsc.load_gather(vis_v, [si])
            v = jnp.where(g == 0.0, sv, jnp.float32(_BIG))
            lt = v < bv
            bv = jnp.where(lt, v, bv)
            bi = jnp.where(lt, si, bi)
        m = jnp.min(bv)
        idx_f = jnp.min(jnp.where(bv == m, bi, _INF_I))

        def fast():
            return idx_f

        def slow():
            pltpu.sync_copy(dist_hbm.at[b, point], row_v)

            @plsc.parallel_loop(0, nchunks, unroll=8,
                                carry=(jnp.full((_L,), 3e6, jnp.float32),
                                       jnp.zeros((_L,), jnp.int32)))
            def fcarry(k, carry):
                fv, fi = carry
                off = k * _L
                v = row_v[pl.ds(off, _L)]
                p = vis_v[pl.ds(off, _L)]
                v = jnp.where(p != 0.0, jnp.float32(_BIG), v)
                flt = v < fv
                return (jnp.where(flt, v, fv),
                        jnp.where(flt, lanes + off, fi))

            fv, fi = fcarry
            fm = jnp.min(fv)
            return jnp.min(jnp.where(fv == fm, fi, _INF_I))

        idx = lax.cond(idx_f < _INF_I, fast, slow)

        idx_vec = jnp.full((_L,), idx, jnp.int32)
        lane0 = lanes == 0
        plsc.store_scatter(vis_v, [idx_vec], big_v, mask=lane0)
        wr = jnp.logical_and(lane0, j < limit)
        plsc.store_scatter(pred_v, [jnp.full((_L,), j, jnp.int32)], idx_vec,
                           mask=wr)
        return idx

    lax.fori_loop(0, n, step, start, unroll=2)
    pltpu.sync_copy(pred_v, out_hbm.at[b])


def kernel(distance, mask, start_idx, pad_value):
    B, N, _ = distance.shape
    assert B == 32 and N % _L == 0 and N % _RPB == 0

    penalty = jnp.where(mask, jnp.float32(_BIG), jnp.float32(0.0))
    limit = (N - jnp.sum(mask.astype(jnp.int32), axis=1)).astype(jnp.int32)
    params = jnp.zeros((B, _L), jnp.int32)
    params = params.at[:, 0].set(start_idx.astype(jnp.int32))
    params = params.at[:, 1].set(limit)
    predinit = jnp.full((B, N), pad_value, jnp.int32)

    mesh = plsc.VectorSubcoreMesh(core_axis_name="c", subcore_axis_name="s")
    run = pl.kernel(
        _greedy_body,
        out_type=jax.ShapeDtypeStruct((B, N), jnp.int32),
        mesh=mesh,
        compiler_params=pltpu.CompilerParams(needs_layout_passes=False),
        scratch_types=[
            pltpu.VMEM((N * _STRIDE,), jnp.float32),
            pltpu.VMEM((N * _STRIDE,), jnp.int32),
            pltpu.VMEM((N,), jnp.int32),
            pltpu.VMEM((N,), jnp.float32),
            pltpu.VMEM((N,), jnp.int32),
            pltpu.VMEM((N,), jnp.float32),
            pltpu.VMEM((_L,), jnp.int32),
            pltpu.VMEM((2, _RPB, N), jnp.float32),
            pltpu.SemaphoreType.DMA((2,)),
        ],
    )
    preds = run(distance, params, penalty, predinit)
    return preds, limit
